# Initial kernel scaffold; baseline (speedup 1.0000x reference)
import jax
import jax.numpy as jnp
from jax.experimental import pallas as pl


def kernel(x, edge_index, pairs, W_l, W_r, att, bias, gamma, beta, running_mean, running_var, W1, b1, W2, b2, W3, b3):
    raise NotImplementedError("write your pallas kernel here")


# speedup vs baseline: 1.3686x; 1.3686x over previous
import functools

import jax
import jax.numpy as jnp
from jax.experimental import pallas as pl
from jax.experimental.pallas import tpu as pltpu


def _node_transform_body(x_ref, wl_ref, wr_ref, xl_ref, xr_ref):
    x = x_ref[...]
    xl_ref[...] = jnp.dot(x, wl_ref[...], preferred_element_type=jnp.float32)
    xr_ref[...] = jnp.dot(x, wr_ref[...], preferred_element_type=jnp.float32)


def _node_transform(x, W_l, W_r):
    N, F = x.shape
    H = W_l.shape[1]
    BN = 1000
    grid = (N // BN,)
    return pl.pallas_call(
        _node_transform_body,
        grid=grid,
        in_specs=[
            pl.BlockSpec((BN, F), lambda i: (i, 0)),
            pl.BlockSpec((F, H), lambda i: (0, 0)),
            pl.BlockSpec((F, H), lambda i: (0, 0)),
        ],
        out_specs=[
            pl.BlockSpec((BN, H), lambda i: (i, 0)),
            pl.BlockSpec((BN, H), lambda i: (i, 0)),
        ],
        out_shape=[
            jax.ShapeDtypeStruct((N, H), jnp.float32),
            jax.ShapeDtypeStruct((N, H), jnp.float32),
        ],
    )(x, W_l, W_r)


def _mlp_body(pel_ref, per_ref, w1a_ref, w1b_ref, b1_ref, w2_ref, b2_ref,
              w3_ref, b3_ref, out_ref):
    z1 = jnp.dot(pel_ref[...], w1a_ref[...], preferred_element_type=jnp.float32)
    z1 += jnp.dot(per_ref[...], w1b_ref[...], preferred_element_type=jnp.float32)
    z1 = jnp.maximum(z1 + b1_ref[...], 0.0)
    z2 = jnp.dot(z1, w2_ref[...], preferred_element_type=jnp.float32)
    z2 = jnp.maximum(z2 + b2_ref[...], 0.0)
    z3 = jnp.dot(z2, w3_ref[...], preferred_element_type=jnp.float32)
    out_ref[...] = jax.nn.sigmoid(z3 + b3_ref[...])


def _pair_mlp(pe_l, pe_r, W1, b1, W2, b2, W3, b3):
    P, H = pe_l.shape
    D1 = W1.shape[1]
    D2 = W2.shape[1]
    W1a = W1[:H]
    W1b = W1[H:]
    b1r = b1.reshape(1, -1)
    b2r = b2.reshape(1, -1)
    W3p = jnp.pad(W3, ((0, 0), (0, 127)))
    b3r = jnp.pad(b3.reshape(1, 1), ((0, 0), (0, 127)))
    BP = 2048
    grid = (P // BP,)
    out = pl.pallas_call(
        _mlp_body,
        grid=grid,
        in_specs=[
            pl.BlockSpec((BP, H), lambda i: (i, 0)),
            pl.BlockSpec((BP, H), lambda i: (i, 0)),
            pl.BlockSpec((H, D1), lambda i: (0, 0)),
            pl.BlockSpec((H, D1), lambda i: (0, 0)),
            pl.BlockSpec((1, D1), lambda i: (0, 0)),
            pl.BlockSpec((D1, D2), lambda i: (0, 0)),
            pl.BlockSpec((1, D2), lambda i: (0, 0)),
            pl.BlockSpec((D2, 128), lambda i: (0, 0)),
            pl.BlockSpec((1, 128), lambda i: (0, 0)),
        ],
        out_specs=pl.BlockSpec((BP, 128), lambda i: (i, 0)),
        out_shape=jax.ShapeDtypeStruct((P, 128), jnp.float32),
    )(pe_l, pe_r, W1a, W1b, b1r, W2, b2r, W3p, b3r)
    return out[:, :1]


def _edge_phase_xla(x_l, x_r, src, dst, att, bias):
    N = x_l.shape[0]
    e = jax.nn.leaky_relu(x_l[src] + x_r[dst], negative_slope=0.2)
    w = jnp.exp(e @ att)
    denom = jax.ops.segment_sum(w, dst, num_segments=N)
    num = jax.ops.segment_sum(x_l[src] * w[:, None], dst, num_segments=N)
    return num / denom[:, None] + bias


def kernel(x, edge_index, pairs, W_l, W_r, att, bias, gamma, beta,
           running_mean, running_var, W1, b1, W2, b2, W3, b3):
    N = x.shape[0]
    loop = jnp.arange(N, dtype=edge_index.dtype)
    src = jnp.concatenate([edge_index[0], loop])
    dst = jnp.concatenate([edge_index[1], loop])

    x_l, x_r = _node_transform(x, W_l, W_r)
    h = jnp.maximum(_edge_phase_xla(x_l, x_r, src, dst, att, bias), 0.0)
    pe_l = h[pairs[:, 0]]
    pe_r = h[pairs[:, 1]]
    return _pair_mlp(pe_l, pe_r, W1, b1, W2, b2, W3, b3)


# speedup vs baseline: 3.4737x; 2.5382x over previous
import functools

import jax
import jax.numpy as jnp
from jax import lax
from jax.experimental import pallas as pl
from jax.experimental.pallas import tpu as pltpu
from jax.experimental.pallas import tpu_sc as plsc

N = 10000
NPAD = 10240
RSZ = 64
NRANGE = 157
NBUCK = 160
BCAP = 128
NW = 32
H = 512
HV = H // 16

_SC_MESH = dict(
    mesh=plsc.VectorSubcoreMesh(core_axis_name="c", subcore_axis_name="s"),
    compiler_params=pltpu.CompilerParams(needs_layout_passes=False),
)


def _wid():
    return lax.axis_index("s") * 2 + lax.axis_index("c")


def _node_transform_body(x_ref, wl_ref, wr_ref, xl_ref, xr_ref):
    x = x_ref[...]
    xl_ref[...] = jnp.dot(x, wl_ref[...], preferred_element_type=jnp.float32)
    xr_ref[...] = jnp.dot(x, wr_ref[...], preferred_element_type=jnp.float32)


def _node_transform(x_pad, W_l, W_r):
    n, f = x_pad.shape
    h = W_l.shape[1]
    bn = 1024
    return pl.pallas_call(
        _node_transform_body,
        grid=(n // bn,),
        in_specs=[
            pl.BlockSpec((bn, f), lambda i: (i, 0)),
            pl.BlockSpec((f, h), lambda i: (0, 0)),
            pl.BlockSpec((f, h), lambda i: (0, 0)),
        ],
        out_specs=[
            pl.BlockSpec((bn, h), lambda i: (i, 0)),
            pl.BlockSpec((bn, h), lambda i: (i, 0)),
        ],
        out_shape=[
            jax.ShapeDtypeStruct((n, h), jnp.float32),
            jax.ShapeDtypeStruct((n, h), jnp.float32),
        ],
    )(x_pad, W_l, W_r)


def _make_bucketize(epw):
    nchunk = epw // 16

    @functools.partial(
        pl.kernel,
        out_type=[
            jax.ShapeDtypeStruct((NW, NBUCK, BCAP), jnp.int32),
            jax.ShapeDtypeStruct((NW, NBUCK), jnp.int32),
        ],
        scratch_types=[
            pltpu.VMEM((epw,), jnp.int32),
            pltpu.VMEM((epw,), jnp.int32),
            pltpu.VMEM((1, NBUCK, BCAP), jnp.int32),
            pltpu.VMEM((1, NBUCK), jnp.int32),
        ],
        **_SC_MESH,
    )
    def bucketize(src_hbm, dst_hbm, buckets_hbm, counts_hbm,
                  sbuf, dbuf, buck, wptr):
        wid = _wid()
        col = lax.iota(jnp.int32, 16)
        z16 = jnp.zeros((16,), jnp.int32)

        for i in range(NBUCK // 16):
            plsc.store_scatter(wptr, [z16, i * 16 + col], z16)

        base = wid * epw
        pltpu.sync_copy(src_hbm.at[pl.ds(base, epw)], sbuf)
        pltpu.sync_copy(dst_hbm.at[pl.ds(base, epw)], dbuf)

        def chunk_body(c, _):
            off = c * 16
            d = dbuf[pl.ds(off, 16)]
            s = sbuf[pl.ds(off, 16)]
            rid = lax.shift_right_logical(d, 6)
            valid = rid < NRANGE
            ridc = jnp.minimum(rid, NBUCK - 1)
            packed = lax.shift_left(s, 6) | (d & 63)
            rank, _last = plsc.scan_count(ridc, mask=valid)
            wp = plsc.load_gather(wptr, [z16, ridc], mask=valid)
            pos = jnp.minimum(wp + rank - 1, BCAP - 1)
            plsc.store_scatter(buck, [z16, ridc, pos], packed, mask=valid)
            plsc.store_scatter(wptr, [z16, ridc],
                               jnp.minimum(pos + 1, BCAP),
                               mask=valid & _last)
            return 0

        lax.fori_loop(0, nchunk, chunk_body, 0)
        pltpu.sync_copy(buck, buckets_hbm.at[pl.ds(wid, 1)])
        pltpu.sync_copy(wptr, counts_hbm.at[pl.ds(wid, 1)])

    return bucketize


@functools.partial(
    pl.kernel,
    out_type=jax.ShapeDtypeStruct((NPAD, H), jnp.float32),
    scratch_types=[
        pltpu.VMEM((RSZ, H), jnp.float32),
        pltpu.VMEM((RSZ, H), jnp.float32),
        pltpu.VMEM((64, H), jnp.float32),
        pltpu.VMEM((H,), jnp.float32),
        pltpu.VMEM((NW, 1, BCAP), jnp.int32),
        pltpu.VMEM((NW, NBUCK), jnp.int32),
        pltpu.VMEM((NW * BCAP,), jnp.int32),
        pltpu.VMEM((64,), jnp.int32),
        pltpu.VMEM((RSZ,), jnp.float32),
        pltpu.SemaphoreType.DMA,
    ],
    **_SC_MESH,
)
def _aggregate(xl_hbm, xr_hbm, att_hbm, buckets_hbm, counts_hbm, zeros_hbm,
               h_hbm, acc, xrc, rows, attv, buck, cnts, plist, sidx,
               denomv, sem):
    wid = _wid()
    col = lax.iota(jnp.int32, 16)
    l15 = jnp.full((16,), 15, jnp.int32)

    pltpu.sync_copy(att_hbm, attv)
    pltpu.sync_copy(counts_hbm, cnts)

    def process_range(r):
        pltpu.sync_copy(zeros_hbm, acc)
        for i in range(RSZ // 16):
            denomv[pl.ds(i * 16, 16)] = jnp.zeros((16,), jnp.float32)
        pltpu.sync_copy(xr_hbm.at[pl.ds(r * RSZ, RSZ)], xrc)
        pltpu.sync_copy(buckets_hbm.at[:, pl.ds(r, 1), :], buck)

        rlane = r & 15
        rslice = lax.shift_left(lax.shift_right_logical(r, 4), 4)
        wpos = jnp.int32(0)
        for k in range(NW):
            cvec = cnts[k, pl.ds(rslice, 16)]
            cnt_k = jnp.sum(jnp.where(col == rlane, cvec, 0))

            def seg_body(i, wp, k=k, cnt_k=cnt_k):
                vals = buck[k, 0, pl.ds(i * 16, 16)]
                m = i * 16 + col < cnt_k
                plsc.store_compressed(plist.at[pl.ds(wp, 16)], vals, mask=m)
                return wp + jnp.minimum(cnt_k - i * 16, 16)

            wpos = lax.fori_loop(0, (cnt_k + 15) // 16, seg_body, wpos)
        total = wpos

        def batch_body(b, _):
            base = b * 64
            for i in range(4):
                pv = plist[pl.ds(base + i * 16, 16)]
                m = base + i * 16 + col < total
                sidx[pl.ds(i * 16, 16)] = jnp.where(
                    m, lax.shift_right_logical(pv, 6), 0)
            pltpu.async_copy(xl_hbm.at[sidx], rows, sem).wait()
            ncur = jnp.minimum(total - base, 64)

            def edge_body(e, _):
                eb = base + e
                pvv = plist[pl.ds(lax.shift_left(
                    lax.shift_right_logical(eb, 4), 4), 16)]
                pk = jnp.take(pvv, jnp.full((16,), eb & 15, jnp.int32))
                ldst = pk & 63
                erow = jnp.full((16,), e, jnp.int32)
                pacc = jnp.zeros((16,), jnp.float32)
                for j in range(HV):
                    cj = j * 16 + col
                    xlj = plsc.load_gather(rows, [erow, cj])
                    xrj = plsc.load_gather(xrc, [ldst, cj])
                    t = xlj + xrj
                    lr = jnp.maximum(t, 0.2 * t)
                    pacc = pacc + attv[pl.ds(j * 16, 16)] * lr
                cs = plsc.cumsum(pacc)
                w = jnp.exp(jnp.take(cs, l15))
                plsc.addupdate_scatter(denomv, [ldst], w, mask=col < 1)
                for j in range(HV):
                    cj = j * 16 + col
                    xlj = plsc.load_gather(rows, [erow, cj])
                    plsc.addupdate_scatter(acc, [ldst, cj], w * xlj)
                return 0

            lax.fori_loop(0, ncur, edge_body, 0)
            return 0

        lax.fori_loop(0, (total + 63) // 64, batch_body, 0)

        for i in range(RSZ // 16):
            dv = denomv[pl.ds(i * 16, 16)]
            denomv[pl.ds(i * 16, 16)] = 1.0 / (dv + 1e-16)

        def fin_body(n, _):
            nv = jnp.full((16,), n, jnp.int32)
            iv = plsc.load_gather(denomv, [nv])
            for j in range(HV):
                cj = j * 16 + col
                v = plsc.load_gather(acc, [nv, cj]) * iv
                plsc.store_scatter(acc, [nv, cj], v)
            return 0

        lax.fori_loop(0, RSZ, fin_body, 0)
        pltpu.sync_copy(acc, h_hbm.at[pl.ds(r * RSZ, RSZ)])

    def range_body(p, _):
        r = wid + p * NW

        @pl.when(r < NRANGE)
        def _():
            process_range(r)
        return 0

    lax.fori_loop(0, (NRANGE + NW - 1) // NW, range_body, 0)


def _make_pair_gather(npairs):
    ppw = npairs // NW
    gb = 128

    @functools.partial(
        pl.kernel,
        out_type=[
            jax.ShapeDtypeStruct((npairs, H), jnp.float32),
            jax.ShapeDtypeStruct((npairs, H), jnp.float32),
        ],
        scratch_types=[
            pltpu.VMEM((ppw,), jnp.int32),
            pltpu.VMEM((gb, H), jnp.float32),
            pltpu.SemaphoreType.DMA,
        ],
        **_SC_MESH,
    )
    def pair_gather(h_hbm, p0_hbm, p1_hbm, pel_hbm, per_hbm,
                    idxv, rowbuf, sem):
        wid = _wid()
        base = wid * ppw
        for p_hbm, out_hbm in ((p0_hbm, pel_hbm), (p1_hbm, per_hbm)):
            pltpu.sync_copy(p_hbm.at[pl.ds(base, ppw)], idxv)
            for b in range(ppw // gb):
                pltpu.async_copy(
                    h_hbm.at[idxv.at[pl.ds(b * gb, gb)]], rowbuf, sem
                ).wait()
                pltpu.sync_copy(
                    rowbuf, out_hbm.at[pl.ds(base + b * gb, gb)])

    return pair_gather


def _mlp_body(pel_ref, per_ref, bias_ref, w1a_ref, w1b_ref, b1_ref,
              w2_ref, b2_ref, w3_ref, b3_ref, out_ref):
    pel = jnp.maximum(pel_ref[...] + bias_ref[...], 0.0)
    per = jnp.maximum(per_ref[...] + bias_ref[...], 0.0)
    z1 = jnp.dot(pel, w1a_ref[...], preferred_element_type=jnp.float32)
    z1 += jnp.dot(per, w1b_ref[...], preferred_element_type=jnp.float32)
    z1 = jnp.maximum(z1 + b1_ref[...], 0.0)
    z2 = jnp.dot(z1, w2_ref[...], preferred_element_type=jnp.float32)
    z2 = jnp.maximum(z2 + b2_ref[...], 0.0)
    z3 = jnp.dot(z2, w3_ref[...], preferred_element_type=jnp.float32)
    out_ref[...] = jax.nn.sigmoid(z3 + b3_ref[...])


def _pair_mlp(pe_l, pe_r, bias, W1, b1, W2, b2, W3, b3):
    p, h = pe_l.shape
    d1 = W1.shape[1]
    d2 = W2.shape[1]
    W1a = W1[:h]
    W1b = W1[h:]
    biasr = bias.reshape(1, -1)
    b1r = b1.reshape(1, -1)
    b2r = b2.reshape(1, -1)
    W3p = jnp.pad(W3, ((0, 0), (0, 127)))
    b3r = jnp.pad(b3.reshape(1, 1), ((0, 0), (0, 127)))
    bp = 2048
    out = pl.pallas_call(
        _mlp_body,
        grid=(p // bp,),
        in_specs=[
            pl.BlockSpec((bp, h), lambda i: (i, 0)),
            pl.BlockSpec((bp, h), lambda i: (i, 0)),
            pl.BlockSpec((1, h), lambda i: (0, 0)),
            pl.BlockSpec((h, d1), lambda i: (0, 0)),
            pl.BlockSpec((h, d1), lambda i: (0, 0)),
            pl.BlockSpec((1, d1), lambda i: (0, 0)),
            pl.BlockSpec((d1, d2), lambda i: (0, 0)),
            pl.BlockSpec((1, d2), lambda i: (0, 0)),
            pl.BlockSpec((d2, 128), lambda i: (0, 0)),
            pl.BlockSpec((1, 128), lambda i: (0, 0)),
        ],
        out_specs=pl.BlockSpec((bp, 128), lambda i: (i, 0)),
        out_shape=jax.ShapeDtypeStruct((p, 128), jnp.float32),
    )(pe_l, pe_r, biasr, W1a, W1b, b1r, W2, b2r, W3p, b3r)
    return out[:, :1]


def kernel(x, edge_index, pairs, W_l, W_r, att, bias, gamma, beta,
           running_mean, running_var, W1, b1, W2, b2, W3, b3):
    n = x.shape[0]
    e2 = edge_index.shape[1] + n
    epw = ((e2 + NW * 16 - 1) // (NW * 16)) * 16
    epad = epw * NW

    loop = jnp.arange(n, dtype=jnp.int32)
    src = jnp.concatenate(
        [edge_index[0].astype(jnp.int32), loop,
         jnp.zeros((epad - e2,), jnp.int32)])
    dst = jnp.concatenate(
        [edge_index[1].astype(jnp.int32), loop,
         jnp.full((epad - e2,), 1 << 20, jnp.int32)])

    x_pad = jnp.pad(x, ((0, NPAD - n), (0, 0)))
    x_l, x_r = _node_transform(x_pad, W_l, W_r)

    buckets, counts = _make_bucketize(epw)(src, dst)
    zeros = jnp.zeros((RSZ, H), jnp.float32)
    h = _aggregate(x_l, x_r, att, buckets, counts, zeros)

    p0 = pairs[:, 0].astype(jnp.int32)
    p1 = pairs[:, 1].astype(jnp.int32)
    pe_l, pe_r = _make_pair_gather(pairs.shape[0])(h, p0, p1)
    return _pair_mlp(pe_l, pe_r, bias, W1, b1, W2, b2, W3, b3)


# speedup vs baseline: 3.9372x; 1.1334x over previous
import functools

import jax
import jax.numpy as jnp
from jax import lax
from jax.experimental import pallas as pl
from jax.experimental.pallas import tpu as pltpu
from jax.experimental.pallas import tpu_sc as plsc

N = 10000
NPAD = 10240
RSZ = 64
NRANGE = 157
NBUCK = 160
BCAP = 128
NW = 32
H = 512
HV = H // 16

_SC_MESH = dict(
    mesh=plsc.VectorSubcoreMesh(core_axis_name="c", subcore_axis_name="s"),
    compiler_params=pltpu.CompilerParams(needs_layout_passes=False),
)


def _wid():
    return lax.axis_index("s") * 2 + lax.axis_index("c")


def _node_transform_body(x_ref, wl_ref, wr_ref, xl_ref, xr_ref):
    x = x_ref[...]
    xl_ref[...] = jnp.dot(x, wl_ref[...], preferred_element_type=jnp.float32)
    xr_ref[...] = jnp.dot(x, wr_ref[...], preferred_element_type=jnp.float32)


def _node_transform(x_pad, W_l, W_r):
    n, f = x_pad.shape
    h = W_l.shape[1]
    bn = 1024
    return pl.pallas_call(
        _node_transform_body,
        grid=(n // bn,),
        in_specs=[
            pl.BlockSpec((bn, f), lambda i: (i, 0)),
            pl.BlockSpec((f, h), lambda i: (0, 0)),
            pl.BlockSpec((f, h), lambda i: (0, 0)),
        ],
        out_specs=[
            pl.BlockSpec((bn, h), lambda i: (i, 0)),
            pl.BlockSpec((bn, h), lambda i: (i, 0)),
        ],
        out_shape=[
            jax.ShapeDtypeStruct((n, h), jnp.float32),
            jax.ShapeDtypeStruct((n, h), jnp.float32),
        ],
    )(x_pad, W_l, W_r)


def _make_bucketize(epw):
    nchunk = epw // 16

    @functools.partial(
        pl.kernel,
        out_type=[
            jax.ShapeDtypeStruct((NW, NBUCK, BCAP), jnp.int32),
            jax.ShapeDtypeStruct((NW, NBUCK), jnp.int32),
        ],
        scratch_types=[
            pltpu.VMEM((epw,), jnp.int32),
            pltpu.VMEM((epw,), jnp.int32),
            pltpu.VMEM((1, NBUCK, BCAP), jnp.int32),
            pltpu.VMEM((1, NBUCK), jnp.int32),
        ],
        **_SC_MESH,
    )
    def bucketize(src_hbm, dst_hbm, buckets_hbm, counts_hbm,
                  sbuf, dbuf, buck, wptr):
        wid = _wid()
        col = lax.iota(jnp.int32, 16)
        z16 = jnp.zeros((16,), jnp.int32)

        for i in range(NBUCK // 16):
            plsc.store_scatter(wptr, [z16, i * 16 + col], z16)

        base = wid * epw
        pltpu.sync_copy(src_hbm.at[pl.ds(base, epw)], sbuf)
        pltpu.sync_copy(dst_hbm.at[pl.ds(base, epw)], dbuf)

        def chunk_body(c, _):
            off = c * 16
            d = dbuf[pl.ds(off, 16)]
            s = sbuf[pl.ds(off, 16)]
            rid = lax.shift_right_logical(d, 6)
            valid = rid < NRANGE
            ridc = jnp.minimum(rid, NBUCK - 1)
            packed = lax.shift_left(s, 6) | (d & 63)
            rank, _last = plsc.scan_count(ridc, mask=valid)
            wp = plsc.load_gather(wptr, [z16, ridc], mask=valid)
            pos = jnp.minimum(wp + rank - 1, BCAP - 1)
            plsc.store_scatter(buck, [z16, ridc, pos], packed, mask=valid)
            plsc.store_scatter(wptr, [z16, ridc],
                               jnp.minimum(pos + 1, BCAP),
                               mask=valid & _last)
            return 0

        lax.fori_loop(0, nchunk, chunk_body, 0)
        pltpu.sync_copy(buck, buckets_hbm.at[pl.ds(wid, 1)])
        pltpu.sync_copy(wptr, counts_hbm.at[pl.ds(wid, 1)])

    return bucketize


GB = 32


@functools.partial(
    pl.kernel,
    out_type=jax.ShapeDtypeStruct((NPAD * H,), jnp.float32),
    scratch_types=[
        pltpu.VMEM((RSZ * H,), jnp.float32),
        pltpu.VMEM((RSZ * H,), jnp.float32),
        pltpu.VMEM((GB, H), jnp.float32),
        pltpu.VMEM((GB, H), jnp.float32),
        pltpu.VMEM((H,), jnp.float32),
        pltpu.VMEM((NW, 1, BCAP), jnp.int32),
        pltpu.VMEM((NW, NBUCK), jnp.int32),
        pltpu.VMEM((NW * BCAP,), jnp.int32),
        pltpu.VMEM((GB,), jnp.int32),
        pltpu.VMEM((GB,), jnp.int32),
        pltpu.VMEM((RSZ,), jnp.float32),
        pltpu.SemaphoreType.DMA,
        pltpu.SemaphoreType.DMA,
    ],
    **_SC_MESH,
)
def _aggregate(xl_hbm, xr_hbm, att_hbm, buckets_hbm, counts_hbm, zeros_hbm,
               h_hbm, acc, xrc, rows0, rows1, attv, buck, cnts, plist,
               sidx0, sidx1, denomv, sem0, sem1):
    wid = _wid()
    col = lax.iota(jnp.int32, 16)
    l15 = jnp.full((16,), 15, jnp.int32)
    bufs = ((sidx0, rows0, sem0), (sidx1, rows1, sem1))

    pltpu.sync_copy(att_hbm, attv)
    pltpu.sync_copy(counts_hbm, cnts)

    def process_range(r):
        pltpu.sync_copy(zeros_hbm, acc)
        for i in range(RSZ // 16):
            denomv[pl.ds(i * 16, 16)] = jnp.zeros((16,), jnp.float32)
        pltpu.sync_copy(xr_hbm.at[pl.ds(r * (RSZ * H), RSZ * H)], xrc)
        pltpu.sync_copy(buckets_hbm.at[:, pl.ds(r, 1), :], buck)

        rlane = r & 15
        rslice = lax.shift_left(lax.shift_right_logical(r, 4), 4)
        wpos = jnp.int32(0)
        for k in range(NW):
            cvec = cnts[k, pl.ds(rslice, 16)]
            cnt_k = jnp.sum(jnp.where(col == rlane, cvec, 0))

            def seg_body(i, wp, k=k, cnt_k=cnt_k):
                vals = buck[k, 0, pl.ds(i * 16, 16)]
                m = i * 16 + col < cnt_k
                plsc.store_compressed(plist.at[pl.ds(wp, 16)], vals, mask=m)
                return wp + jnp.minimum(cnt_k - i * 16, 16)

            wpos = lax.fori_loop(0, (cnt_k + 15) // 16, seg_body, wpos)
        total = wpos
        nb = (total + GB - 1) // GB

        def fill(b, sidx, rows, sem):
            base = b * GB
            for i in range(GB // 16):
                pv = plist[pl.ds(base + i * 16, 16)]
                m = base + i * 16 + col < total
                sidx[pl.ds(i * 16, 16)] = jnp.where(
                    m, lax.shift_right_logical(pv, 6), 0)
            pltpu.async_copy(xl_hbm.at[sidx], rows, sem)

        def process(b, rows):
            base = b * GB
            ncur = jnp.minimum(total - base, GB)

            def _edge(e, _):
                eb = base + e
                pvv = plist[pl.ds(lax.shift_left(
                    lax.shift_right_logical(eb, 4), 4), 16)]
                pk = jnp.take(pvv, jnp.full((16,), eb & 15, jnp.int32))
                ldst = pk & 63
                doff = lax.reduce_max(ldst, (0,)) * H
                pacc = jnp.zeros((16,), jnp.float32)
                for j in range(HV):
                    xlj = rows[e, pl.ds(j * 16, 16)]
                    xrj = xrc[pl.ds(doff + j * 16, 16)]
                    t = xlj + xrj
                    lr = jnp.maximum(t, 0.2 * t)
                    pacc = pacc + attv[pl.ds(j * 16, 16)] * lr
                cs = plsc.cumsum(pacc)
                w = jnp.exp(jnp.take(cs, l15))
                plsc.addupdate_scatter(denomv, [ldst], w, mask=col < 1)
                for j in range(HV):
                    xlj = rows[e, pl.ds(j * 16, 16)]
                    plsc.addupdate(acc.at[pl.ds(doff + j * 16, 16)],
                                   w * xlj)
                return 0

            lax.fori_loop(0, ncur, _edge, 0)

        fill(0, *bufs[0])

        def group_body(g, _):
            for ph in range(2):
                b = g * 2 + ph
                sidx, rows, sem = bufs[ph]
                nsidx, nrows, nsem = bufs[1 - ph]

                @pl.when(b < nb)
                def _():
                    @pl.when(b + 1 < nb)
                    def _():
                        fill(b + 1, nsidx, nrows, nsem)
                    pltpu.make_async_copy(
                        xl_hbm.at[sidx], rows, sem).wait()
                    process(b, rows)
            return 0

        lax.fori_loop(0, (nb + 1) // 2, group_body, 0)

        for i in range(RSZ // 16):
            dv = denomv[pl.ds(i * 16, 16)]
            denomv[pl.ds(i * 16, 16)] = 1.0 / (dv + 1e-16)

        def fin_body(n, _):
            iv = plsc.load_gather(denomv, [jnp.full((16,), n, jnp.int32)])
            noff = n * H
            for j in range(HV):
                off = noff + j * 16
                acc[pl.ds(off, 16)] = acc[pl.ds(off, 16)] * iv
            return 0

        lax.fori_loop(0, RSZ, fin_body, 0)
        pltpu.sync_copy(acc, h_hbm.at[pl.ds(r * (RSZ * H), RSZ * H)])

    def range_body(p, _):
        r = wid + p * NW

        @pl.when(r < NRANGE)
        def _():
            process_range(r)
        return 0

    lax.fori_loop(0, (NRANGE + NW - 1) // NW, range_body, 0)


def _make_pair_gather(npairs):
    ppw = npairs // NW
    gb = 128

    @functools.partial(
        pl.kernel,
        out_type=[
            jax.ShapeDtypeStruct((npairs, H), jnp.float32),
            jax.ShapeDtypeStruct((npairs, H), jnp.float32),
        ],
        scratch_types=[
            pltpu.VMEM((ppw,), jnp.int32),
            pltpu.VMEM((gb, H), jnp.float32),
            pltpu.SemaphoreType.DMA,
        ],
        **_SC_MESH,
    )
    def pair_gather(h_hbm, p0_hbm, p1_hbm, pel_hbm, per_hbm,
                    idxv, rowbuf, sem):
        wid = _wid()
        base = wid * ppw
        for p_hbm, out_hbm in ((p0_hbm, pel_hbm), (p1_hbm, per_hbm)):
            pltpu.sync_copy(p_hbm.at[pl.ds(base, ppw)], idxv)
            for b in range(ppw // gb):
                pltpu.async_copy(
                    h_hbm.at[idxv.at[pl.ds(b * gb, gb)]], rowbuf, sem
                ).wait()
                pltpu.sync_copy(
                    rowbuf, out_hbm.at[pl.ds(base + b * gb, gb)])

    return pair_gather


def _mlp_body(pel_ref, per_ref, bias_ref, w1a_ref, w1b_ref, b1_ref,
              w2_ref, b2_ref, w3_ref, b3_ref, out_ref):
    pel = jnp.maximum(pel_ref[...] + bias_ref[...], 0.0)
    per = jnp.maximum(per_ref[...] + bias_ref[...], 0.0)
    z1 = jnp.dot(pel, w1a_ref[...], preferred_element_type=jnp.float32)
    z1 += jnp.dot(per, w1b_ref[...], preferred_element_type=jnp.float32)
    z1 = jnp.maximum(z1 + b1_ref[...], 0.0)
    z2 = jnp.dot(z1, w2_ref[...], preferred_element_type=jnp.float32)
    z2 = jnp.maximum(z2 + b2_ref[...], 0.0)
    z3 = jnp.dot(z2, w3_ref[...], preferred_element_type=jnp.float32)
    out_ref[...] = jax.nn.sigmoid(z3 + b3_ref[...])


def _pair_mlp(pe_l, pe_r, bias, W1, b1, W2, b2, W3, b3):
    p, h = pe_l.shape
    d1 = W1.shape[1]
    d2 = W2.shape[1]
    W1a = W1[:h]
    W1b = W1[h:]
    biasr = bias.reshape(1, -1)
    b1r = b1.reshape(1, -1)
    b2r = b2.reshape(1, -1)
    W3p = jnp.pad(W3, ((0, 0), (0, 127)))
    b3r = jnp.pad(b3.reshape(1, 1), ((0, 0), (0, 127)))
    bp = 2048
    out = pl.pallas_call(
        _mlp_body,
        grid=(p // bp,),
        in_specs=[
            pl.BlockSpec((bp, h), lambda i: (i, 0)),
            pl.BlockSpec((bp, h), lambda i: (i, 0)),
            pl.BlockSpec((1, h), lambda i: (0, 0)),
            pl.BlockSpec((h, d1), lambda i: (0, 0)),
            pl.BlockSpec((h, d1), lambda i: (0, 0)),
            pl.BlockSpec((1, d1), lambda i: (0, 0)),
            pl.BlockSpec((d1, d2), lambda i: (0, 0)),
            pl.BlockSpec((1, d2), lambda i: (0, 0)),
            pl.BlockSpec((d2, 128), lambda i: (0, 0)),
            pl.BlockSpec((1, 128), lambda i: (0, 0)),
        ],
        out_specs=pl.BlockSpec((bp, 128), lambda i: (i, 0)),
        out_shape=jax.ShapeDtypeStruct((p, 128), jnp.float32),
    )(pe_l, pe_r, biasr, W1a, W1b, b1r, W2, b2r, W3p, b3r)
    return out[:, :1]


def kernel(x, edge_index, pairs, W_l, W_r, att, bias, gamma, beta,
           running_mean, running_var, W1, b1, W2, b2, W3, b3):
    n = x.shape[0]
    e2 = edge_index.shape[1] + n
    epw = ((e2 + NW * 16 - 1) // (NW * 16)) * 16
    epad = epw * NW

    loop = jnp.arange(n, dtype=jnp.int32)
    src = jnp.concatenate(
        [edge_index[0].astype(jnp.int32), loop,
         jnp.zeros((epad - e2,), jnp.int32)])
    dst = jnp.concatenate(
        [edge_index[1].astype(jnp.int32), loop,
         jnp.full((epad - e2,), 1 << 20, jnp.int32)])

    x_pad = jnp.pad(x, ((0, NPAD - n), (0, 0)))
    x_l, x_r = _node_transform(x_pad, W_l, W_r)

    buckets, counts = _make_bucketize(epw)(src, dst)
    zeros = jnp.zeros((RSZ * H,), jnp.float32)
    h = _aggregate(x_l, x_r.reshape(-1), att, buckets, counts, zeros)
    h = h.reshape(NPAD, H)

    p0 = pairs[:, 0].astype(jnp.int32)
    p1 = pairs[:, 1].astype(jnp.int32)
    pe_l, pe_r = _make_pair_gather(pairs.shape[0])(h, p0, p1)
    return _pair_mlp(pe_l, pe_r, bias, W1, b1, W2, b2, W3, b3)


# speedup vs baseline: 6.9508x; 1.7654x over previous
import functools

import jax
import jax.numpy as jnp
from jax import lax
from jax.experimental import pallas as pl
from jax.experimental.pallas import tpu as pltpu
from jax.experimental.pallas import tpu_sc as plsc

N = 10000
NPAD = 10240
RSZ = 64
NRANGE = 157
NBUCK = 160
BCAP = 128
NW = 32
H = 512
HV = H // 16

_SC_MESH = dict(
    mesh=plsc.VectorSubcoreMesh(core_axis_name="c", subcore_axis_name="s"),
    compiler_params=pltpu.CompilerParams(needs_layout_passes=False),
)


def _wid():
    return lax.axis_index("s") * 2 + lax.axis_index("c")


def _node_transform_body(x_ref, wl_ref, wr_ref, xl_ref, xr_ref):
    x = x_ref[...]
    xl_ref[...] = jnp.dot(x, wl_ref[...], preferred_element_type=jnp.float32)
    xr_ref[...] = jnp.dot(x, wr_ref[...], preferred_element_type=jnp.float32)


def _node_transform(x_pad, W_l, W_r):
    n, f = x_pad.shape
    h = W_l.shape[1]
    bn = 1024
    return pl.pallas_call(
        _node_transform_body,
        grid=(n // bn,),
        in_specs=[
            pl.BlockSpec((bn, f), lambda i: (i, 0)),
            pl.BlockSpec((f, h), lambda i: (0, 0)),
            pl.BlockSpec((f, h), lambda i: (0, 0)),
        ],
        out_specs=[
            pl.BlockSpec((bn, h), lambda i: (i, 0)),
            pl.BlockSpec((bn, h), lambda i: (i, 0)),
        ],
        out_shape=[
            jax.ShapeDtypeStruct((n, h), jnp.float32),
            jax.ShapeDtypeStruct((n, h), jnp.float32),
        ],
    )(x_pad, W_l, W_r)


def _make_bucketize(epw):
    nchunk = epw // 16

    @functools.partial(
        pl.kernel,
        out_type=[
            jax.ShapeDtypeStruct((NW, NBUCK, BCAP), jnp.int32),
            jax.ShapeDtypeStruct((NW, NBUCK), jnp.int32),
        ],
        scratch_types=[
            pltpu.VMEM((epw,), jnp.int32),
            pltpu.VMEM((epw,), jnp.int32),
            pltpu.VMEM((1, NBUCK, BCAP), jnp.int32),
            pltpu.VMEM((1, NBUCK), jnp.int32),
        ],
        **_SC_MESH,
    )
    def bucketize(src_hbm, dst_hbm, buckets_hbm, counts_hbm,
                  sbuf, dbuf, buck, wptr):
        wid = _wid()
        col = lax.iota(jnp.int32, 16)
        z16 = jnp.zeros((16,), jnp.int32)

        for i in range(NBUCK // 16):
            plsc.store_scatter(wptr, [z16, i * 16 + col], z16)

        base = wid * epw
        pltpu.sync_copy(src_hbm.at[pl.ds(base, epw)], sbuf)
        pltpu.sync_copy(dst_hbm.at[pl.ds(base, epw)], dbuf)

        def chunk_body(c, _):
            off = c * 16
            d = dbuf[pl.ds(off, 16)]
            s = sbuf[pl.ds(off, 16)]
            rid = lax.shift_right_logical(d, 6)
            valid = rid < NRANGE
            ridc = jnp.minimum(rid, NBUCK - 1)
            packed = lax.shift_left(s, 6) | (d & 63)
            rank, _last = plsc.scan_count(ridc, mask=valid)
            wp = plsc.load_gather(wptr, [z16, ridc], mask=valid)
            pos = jnp.minimum(wp + rank - 1, BCAP - 1)
            plsc.store_scatter(buck, [z16, ridc, pos], packed, mask=valid)
            plsc.store_scatter(wptr, [z16, ridc],
                               jnp.minimum(pos + 1, BCAP),
                               mask=valid & _last)
            return 0

        lax.fori_loop(0, nchunk, chunk_body, 0)
        pltpu.sync_copy(buck, buckets_hbm.at[pl.ds(wid, 1)])
        pltpu.sync_copy(wptr, counts_hbm.at[pl.ds(wid, 1)])

    return bucketize


GB = 32


@functools.partial(
    pl.kernel,
    out_type=jax.ShapeDtypeStruct((NPAD * H,), jnp.float32),
    scratch_types=[
        pltpu.VMEM((RSZ * H,), jnp.float32),
        pltpu.VMEM((RSZ * H,), jnp.float32),
        pltpu.VMEM((GB, H), jnp.float32),
        pltpu.VMEM((GB, H), jnp.float32),
        pltpu.VMEM((H,), jnp.float32),
        pltpu.VMEM((NW, 1, BCAP), jnp.int32),
        pltpu.VMEM((NW, NBUCK), jnp.int32),
        pltpu.VMEM((NW * BCAP,), jnp.int32),
        pltpu.VMEM((GB,), jnp.int32),
        pltpu.VMEM((GB,), jnp.int32),
        pltpu.VMEM((RSZ,), jnp.float32),
        pltpu.SemaphoreType.DMA,
        pltpu.SemaphoreType.DMA,
    ],
    **_SC_MESH,
)
def _aggregate(xl_hbm, xr_hbm, att_hbm, buckets_hbm, counts_hbm, zeros_hbm,
               h_hbm, acc, xrc, rows0, rows1, attv, buck, cnts, plist,
               sidx0, sidx1, denomv, sem0, sem1):
    wid = _wid()
    col = lax.iota(jnp.int32, 16)
    l15 = jnp.full((16,), 15, jnp.int32)
    bufs = ((sidx0, rows0, sem0), (sidx1, rows1, sem1))

    pltpu.sync_copy(att_hbm, attv)
    pltpu.sync_copy(counts_hbm, cnts)

    def process_range(r):
        pltpu.sync_copy(zeros_hbm, acc)
        for i in range(RSZ // 16):
            denomv[pl.ds(i * 16, 16)] = jnp.zeros((16,), jnp.float32)
        pltpu.sync_copy(xr_hbm.at[pl.ds(r * (RSZ * H), RSZ * H)], xrc)
        pltpu.sync_copy(buckets_hbm.at[:, pl.ds(r, 1), :], buck)

        rlane = r & 15
        rslice = lax.shift_left(lax.shift_right_logical(r, 4), 4)
        wpos = jnp.int32(0)
        for k in range(NW):
            cvec = cnts[k, pl.ds(rslice, 16)]
            cnt_k = jnp.sum(jnp.where(col == rlane, cvec, 0))

            def seg_body(i, wp, k=k, cnt_k=cnt_k):
                vals = buck[k, 0, pl.ds(i * 16, 16)]
                m = i * 16 + col < cnt_k
                plsc.store_compressed(plist.at[pl.ds(wp, 16)], vals, mask=m)
                return wp + jnp.minimum(cnt_k - i * 16, 16)

            wpos = lax.fori_loop(0, (cnt_k + 15) // 16, seg_body, wpos)
        total = wpos
        nb = (total + GB - 1) // GB

        def fill(b, sidx, rows, sem):
            base = b * GB
            for i in range(GB // 16):
                pv = plist[pl.ds(base + i * 16, 16)]
                m = base + i * 16 + col < total
                sidx[pl.ds(i * 16, 16)] = jnp.where(
                    m, lax.shift_right_logical(pv, 6), 0)
            pltpu.async_copy(xl_hbm.at[sidx], rows, sem)

        def process(b, rows):
            base = b * GB
            ncur = jnp.minimum(total - base, GB)

            @plsc.parallel_loop(0, ncur, unroll=1)
            def _edge(e):
                eb = base + e
                pvv = plist[pl.ds(lax.shift_left(
                    lax.shift_right_logical(eb, 4), 4), 16)]
                pk = jnp.take(pvv, jnp.full((16,), eb & 15, jnp.int32))
                ldst = pk & 63
                doff = lax.reduce_max(ldst, (0,)) * H
                pacc = jnp.zeros((16,), jnp.float32)
                for j in range(HV):
                    xlj = rows[e, pl.ds(j * 16, 16)]
                    xrj = xrc[pl.ds(doff + j * 16, 16)]
                    t = xlj + xrj
                    lr = jnp.maximum(t, 0.2 * t)
                    pacc = pacc + attv[pl.ds(j * 16, 16)] * lr
                cs = plsc.cumsum(pacc)
                w = jnp.exp(jnp.take(cs, l15))
                plsc.addupdate_scatter(denomv, [ldst], w, mask=col < 1)
                for j in range(HV):
                    xlj = rows[e, pl.ds(j * 16, 16)]
                    plsc.addupdate(acc.at[pl.ds(doff + j * 16, 16)],
                                   w * xlj)

        fill(0, *bufs[0])

        def group_body(g, _):
            for ph in range(2):
                b = g * 2 + ph
                sidx, rows, sem = bufs[ph]
                nsidx, nrows, nsem = bufs[1 - ph]

                @pl.when(b < nb)
                def _():
                    @pl.when(b + 1 < nb)
                    def _():
                        fill(b + 1, nsidx, nrows, nsem)
                    pltpu.make_async_copy(
                        xl_hbm.at[sidx], rows, sem).wait()
                    process(b, rows)
            return 0

        lax.fori_loop(0, (nb + 1) // 2, group_body, 0)

        for i in range(RSZ // 16):
            dv = denomv[pl.ds(i * 16, 16)]
            denomv[pl.ds(i * 16, 16)] = 1.0 / (dv + 1e-16)

        def fin_body(n, _):
            iv = plsc.load_gather(denomv, [jnp.full((16,), n, jnp.int32)])
            noff = n * H
            for j in range(HV):
                off = noff + j * 16
                acc[pl.ds(off, 16)] = acc[pl.ds(off, 16)] * iv
            return 0

        lax.fori_loop(0, RSZ, fin_body, 0)
        pltpu.sync_copy(acc, h_hbm.at[pl.ds(r * (RSZ * H), RSZ * H)])

    def range_body(p, _):
        r = wid + p * NW

        @pl.when(r < NRANGE)
        def _():
            process_range(r)
        return 0

    lax.fori_loop(0, (NRANGE + NW - 1) // NW, range_body, 0)


def _make_pair_gather(npairs):
    ppw = npairs // NW
    gb = 128

    @functools.partial(
        pl.kernel,
        out_type=[
            jax.ShapeDtypeStruct((npairs, H), jnp.float32),
            jax.ShapeDtypeStruct((npairs, H), jnp.float32),
        ],
        scratch_types=[
            pltpu.VMEM((ppw,), jnp.int32),
            pltpu.VMEM((gb, H), jnp.float32),
            pltpu.SemaphoreType.DMA,
        ],
        **_SC_MESH,
    )
    def pair_gather(h_hbm, p0_hbm, p1_hbm, pel_hbm, per_hbm,
                    idxv, rowbuf, sem):
        wid = _wid()
        base = wid * ppw
        for p_hbm, out_hbm in ((p0_hbm, pel_hbm), (p1_hbm, per_hbm)):
            pltpu.sync_copy(p_hbm.at[pl.ds(base, ppw)], idxv)
            for b in range(ppw // gb):
                pltpu.async_copy(
                    h_hbm.at[idxv.at[pl.ds(b * gb, gb)]], rowbuf, sem
                ).wait()
                pltpu.sync_copy(
                    rowbuf, out_hbm.at[pl.ds(base + b * gb, gb)])

    return pair_gather


def _mlp_body(pel_ref, per_ref, bias_ref, w1a_ref, w1b_ref, b1_ref,
              w2_ref, b2_ref, w3_ref, b3_ref, out_ref):
    pel = jnp.maximum(pel_ref[...] + bias_ref[...], 0.0)
    per = jnp.maximum(per_ref[...] + bias_ref[...], 0.0)
    z1 = jnp.dot(pel, w1a_ref[...], preferred_element_type=jnp.float32)
    z1 += jnp.dot(per, w1b_ref[...], preferred_element_type=jnp.float32)
    z1 = jnp.maximum(z1 + b1_ref[...], 0.0)
    z2 = jnp.dot(z1, w2_ref[...], preferred_element_type=jnp.float32)
    z2 = jnp.maximum(z2 + b2_ref[...], 0.0)
    z3 = jnp.dot(z2, w3_ref[...], preferred_element_type=jnp.float32)
    out_ref[...] = jax.nn.sigmoid(z3 + b3_ref[...])


def _pair_mlp(pe_l, pe_r, bias, W1, b1, W2, b2, W3, b3):
    p, h = pe_l.shape
    d1 = W1.shape[1]
    d2 = W2.shape[1]
    W1a = W1[:h]
    W1b = W1[h:]
    biasr = bias.reshape(1, -1)
    b1r = b1.reshape(1, -1)
    b2r = b2.reshape(1, -1)
    W3p = jnp.pad(W3, ((0, 0), (0, 127)))
    b3r = jnp.pad(b3.reshape(1, 1), ((0, 0), (0, 127)))
    bp = 2048
    out = pl.pallas_call(
        _mlp_body,
        grid=(p // bp,),
        in_specs=[
            pl.BlockSpec((bp, h), lambda i: (i, 0)),
            pl.BlockSpec((bp, h), lambda i: (i, 0)),
            pl.BlockSpec((1, h), lambda i: (0, 0)),
            pl.BlockSpec((h, d1), lambda i: (0, 0)),
            pl.BlockSpec((h, d1), lambda i: (0, 0)),
            pl.BlockSpec((1, d1), lambda i: (0, 0)),
            pl.BlockSpec((d1, d2), lambda i: (0, 0)),
            pl.BlockSpec((1, d2), lambda i: (0, 0)),
            pl.BlockSpec((d2, 128), lambda i: (0, 0)),
            pl.BlockSpec((1, 128), lambda i: (0, 0)),
        ],
        out_specs=pl.BlockSpec((bp, 128), lambda i: (i, 0)),
        out_shape=jax.ShapeDtypeStruct((p, 128), jnp.float32),
    )(pe_l, pe_r, biasr, W1a, W1b, b1r, W2, b2r, W3p, b3r)
    return out[:, :1]


def kernel(x, edge_index, pairs, W_l, W_r, att, bias, gamma, beta,
           running_mean, running_var, W1, b1, W2, b2, W3, b3):
    n = x.shape[0]
    e2 = edge_index.shape[1] + n
    epw = ((e2 + NW * 16 - 1) // (NW * 16)) * 16
    epad = epw * NW

    loop = jnp.arange(n, dtype=jnp.int32)
    src = jnp.concatenate(
        [edge_index[0].astype(jnp.int32), loop,
         jnp.zeros((epad - e2,), jnp.int32)])
    dst = jnp.concatenate(
        [edge_index[1].astype(jnp.int32), loop,
         jnp.full((epad - e2,), 1 << 20, jnp.int32)])

    x_pad = jnp.pad(x, ((0, NPAD - n), (0, 0)))
    x_l, x_r = _node_transform(x_pad, W_l, W_r)

    buckets, counts = _make_bucketize(epw)(src, dst)
    zeros = jnp.zeros((RSZ * H,), jnp.float32)
    h = _aggregate(x_l, x_r.reshape(-1), att, buckets, counts, zeros)
    h = h.reshape(NPAD, H)

    p0 = pairs[:, 0].astype(jnp.int32)
    p1 = pairs[:, 1].astype(jnp.int32)
    pe_l, pe_r = _make_pair_gather(pairs.shape[0])(h, p0, p1)
    return _pair_mlp(pe_l, pe_r, bias, W1, b1, W2, b2, W3, b3)
